# Initial kernel scaffold; baseline (speedup 1.0000x reference)
import functools

import jax
import jax.numpy as jnp
from jax import lax
from jax.experimental import pallas as pl
from jax.experimental.pallas import tpu as pltpu
from jax.experimental.pallas import tpu_sc as plsc

N_OPS = 50000
E = 800000
IN_C = 128
OUT_C = 64
HID = 256

R = 512
NPAD = ((N_OPS + R - 1) // R) * R
HALF = NPAD // 2
CH = 128
N_GCH = NPAD // CH
N_ECH = E // CH
NC, NS = 2, 16
TPR = HALF // NS
CPY = 224
NDUMMY = 16

_mesh = plsc.VectorSubcoreMesh(core_axis_name="c", subcore_axis_name="s")


@functools.partial(
    pl.kernel,
    mesh=_mesh,
    out_type=(
        jax.ShapeDtypeStruct((NPAD, OUT_C), jnp.float32),
        jax.ShapeDtypeStruct((NPAD, IN_C), jnp.float32),
        jax.ShapeDtypeStruct((NPAD, IN_C), jnp.float32),
    ),
    scratch_types=[
        pltpu.VMEM((CH,), jnp.int32),
        pltpu.VMEM((CH,), jnp.int32),
        pltpu.VMEM((CH,), jnp.int32),
        pltpu.VMEM((CH, OUT_C), jnp.float32),
        pltpu.VMEM((CH,), jnp.int32),
        pltpu.VMEM((CH, IN_C), jnp.float32),
        pltpu.VMEM((CPY, OUT_C), jnp.float32),
        pltpu.VMEM_SHARED((HALF + NDUMMY, OUT_C), jnp.float32),
        pltpu.SemaphoreType.DMA,
    ],
)
def _sc_kernel(res_hbm, dst_hbm, src_hbm, ops_hbm, preds_hbm, succs_hbm,
               agg_out, gpred_out, gsucc_out,
               draw, sidx, dadj, rbuf, ibuf, obuf, zbuf, shared, sem):
    c = lax.axis_index("c")
    s = lax.axis_index("s")
    gid = s * NC + c

    def _gather(idx_hbm, out_hbm):
        trips = (N_GCH - gid + NC * NS - 1) // (NC * NS)

        def body(j, carry):
            off = pl.multiple_of((gid + j * NC * NS) * CH, CH)
            pltpu.sync_copy(idx_hbm.at[pl.ds(off, CH)], ibuf)
            pltpu.async_copy(ops_hbm.at[ibuf], obuf, sem).wait()
            pltpu.sync_copy(obuf, out_hbm.at[pl.ds(off, CH), :])
            return carry

        lax.fori_loop(0, trips, body, 0)

    _gather(preds_hbm, gpred_out)
    _gather(succs_hbm, gsucc_out)

    def zrow(r, carry):
        for k in range(OUT_C // 16):
            zbuf[r, pl.ds(k * 16, 16)] = jnp.zeros((16,), jnp.float32)
        return carry

    lax.fori_loop(0, CPY, zrow, 0)
    for q in range(TPR // CPY):
        pltpu.sync_copy(zbuf, shared.at[pl.ds(s * TPR + q * CPY, CPY), :])
    plsc.subcore_barrier()

    lo0 = c * HALF
    spill = HALF + s

    def ebody(j, carry):
        off = pl.multiple_of((s + j * NS) * CH, CH)
        pltpu.sync_copy(dst_hbm.at[pl.ds(off, CH)], draw)
        pltpu.sync_copy(src_hbm.at[pl.ds(off, CH)], sidx)
        cp = pltpu.async_copy(res_hbm.at[sidx], rbuf, sem)
        for k in range(CH // 16):
            v = draw[pl.ds(k * 16, 16)] - lo0
            ok = (v >= 0) & (v < HALF)
            dadj[pl.ds(k * 16, 16)] = jnp.where(ok, v, spill)
        cp.wait()
        pltpu.sync_copy(rbuf, shared.at[dadj], add=True)
        return carry

    lax.fori_loop(0, (N_ECH - s + NS - 1) // NS, ebody, 0)
    plsc.subcore_barrier()

    for q in range(TPR // CPY):
        b0 = s * TPR + q * CPY
        pltpu.sync_copy(shared.at[pl.ds(b0, CPY), :], zbuf)
        pltpu.sync_copy(zbuf, agg_out.at[pl.ds(c * HALF + b0, CPY), :])


def _elu(x):
    return jnp.where(x > 0, x, jnp.expm1(x))


def _mlp3(x, w):
    h = _elu(jnp.dot(x, w[0][...], preferred_element_type=jnp.float32) + w[1][...])
    h = _elu(jnp.dot(h, w[2][...], preferred_element_type=jnp.float32) + w[3][...])
    return jnp.dot(h, w[4][...], preferred_element_type=jnp.float32) + w[5][...]


def _tc_body(gp, gs, op, ag, *rest):
    ws = rest[:30]
    out = rest[30]
    p = _mlp3(gp[...], ws[0:6])
    q = _mlp3(gs[...], ws[6:12])
    m = _mlp3(op[...], ws[12:18])
    a = _mlp3(ag[...], ws[18:24])
    o = _mlp3(jnp.concatenate([p, q, a, m], axis=-1), ws[24:30])
    i = pl.program_id(0)
    rows = i * R + lax.broadcasted_iota(jnp.int32, (R, 1), 0)
    keep = (rows >= 1) & (rows <= N_OPS - 2)
    out[...] = jnp.where(keep, o, 0.0)


def _full_spec(arr):
    return pl.BlockSpec(arr.shape, lambda i: (0,) * arr.ndim)


def _tc_call(gpred, gsucc, ops_pad, agg, ws):
    in_specs = [
        pl.BlockSpec((R, IN_C), lambda i: (i, 0)),
        pl.BlockSpec((R, IN_C), lambda i: (i, 0)),
        pl.BlockSpec((R, IN_C), lambda i: (i, 0)),
        pl.BlockSpec((R, OUT_C), lambda i: (i, 0)),
    ] + [_full_spec(w) for w in ws]
    return pl.pallas_call(
        _tc_body,
        grid=(NPAD // R,),
        in_specs=in_specs,
        out_specs=pl.BlockSpec((R, OUT_C), lambda i: (i, 0)),
        out_shape=jax.ShapeDtypeStruct((NPAD, OUT_C), jnp.float32),
        compiler_params=pltpu.CompilerParams(
            dimension_semantics=("arbitrary",)),
    )(gpred, gsucc, ops_pad, agg, *ws)


def kernel(operations, resources, requirement_edges, preds, succs, params):
    n = operations.shape[0]
    dst = requirement_edges[0].astype(jnp.int32)
    src = requirement_edges[1].astype(jnp.int32)
    preds_pad = jnp.pad(preds.astype(jnp.int32), (0, NPAD - n))
    succs_pad = jnp.pad(succs.astype(jnp.int32), (0, NPAD - n))

    agg, gpred, gsucc = _sc_kernel(
        resources, dst, src, operations, preds_pad, succs_pad)

    ops_pad = jnp.pad(operations, ((0, NPAD - n), (0, 0)))
    ws = []
    for nm in ("pred", "succ", "same", "res", "comb"):
        for j in range(3):
            ws.append(params[f"{nm}_W{j}"])
            ws.append(params[f"{nm}_b{j}"].reshape(1, -1))

    out = _tc_call(gpred, gsucc, ops_pad, agg, ws)
    return out[:n]


# speedup vs baseline: 3.1458x; 3.1458x over previous
import functools

import jax
import jax.numpy as jnp
from jax import lax
from jax.experimental import pallas as pl
from jax.experimental.pallas import tpu as pltpu
from jax.experimental.pallas import tpu_sc as plsc

N_OPS = 50000
E = 800000
IN_C = 128
OUT_C = 64
HID = 256

R = 512
NPAD = ((N_OPS + R - 1) // R) * R
HALF = NPAD // 2
CH = 128
GCH = 64
N_GCH = NPAD // GCH
N_ECH = E // CH
NC, NS = 2, 16
TPR = HALF // NS
CPY = 112
NDUMMY = 16

def _sc_call(*args):
    mesh = plsc.VectorSubcoreMesh(core_axis_name="c", subcore_axis_name="s")
    f = pl.kernel(
        _sc_body,
        mesh=mesh,
        out_type=(
            jax.ShapeDtypeStruct((NPAD, OUT_C), jnp.float32),
            jax.ShapeDtypeStruct((NPAD, IN_C), jnp.float32),
            jax.ShapeDtypeStruct((NPAD, IN_C), jnp.float32),
        ),
        scratch_types=[
            pltpu.VMEM((CH,), jnp.int32),
            pltpu.VMEM((CH,), jnp.int32),
            pltpu.VMEM((CH,), jnp.int32),
            pltpu.VMEM((CH, OUT_C), jnp.float32),
            pltpu.VMEM((GCH,), jnp.int32),
            pltpu.VMEM((GCH, IN_C), jnp.float32),
            pltpu.VMEM((CPY, OUT_C), jnp.float32),
            pltpu.VMEM_SHARED((HALF + NDUMMY, OUT_C), jnp.float32),
            pltpu.SemaphoreType.DMA,
        ],
        compiler_params=pltpu.CompilerParams(use_tc_tiling_on_sc=False),
    )
    return f(*args)


def _sc_body(res_hbm, dst_hbm, src_hbm, ops_hbm, preds_hbm, succs_hbm,
               agg_out, gpred_out, gsucc_out,
               draw, sidx, dadj, rbuf, ibuf, obuf, zbuf, shared, sem):
    c = lax.axis_index("c")
    s = lax.axis_index("s")
    gid = s * NC + c

    def _gather(idx_hbm, out_hbm):
        trips = (N_GCH - gid + NC * NS - 1) // (NC * NS)

        def body(j, carry):
            off = pl.multiple_of((gid + j * NC * NS) * CH, CH)
            pltpu.sync_copy(idx_hbm.at[pl.ds(off, CH)], ibuf)
            pltpu.async_copy(ops_hbm.at[ibuf], obuf, sem).wait()
            pltpu.sync_copy(obuf, out_hbm.at[pl.ds(off, CH), :])
            return carry

        lax.fori_loop(0, trips, body, 0)

    _gather(preds_hbm, gpred_out)
    _gather(succs_hbm, gsucc_out)

    def zrow(r, carry):
        for k in range(OUT_C // 16):
            zbuf[r, pl.ds(k * 16, 16)] = jnp.zeros((16,), jnp.float32)
        return carry

    lax.fori_loop(0, CPY, zrow, 0)
    for q in range(TPR // CPY):
        pltpu.sync_copy(zbuf, shared.at[pl.ds(s * TPR + q * CPY, CPY), :])
    plsc.subcore_barrier()

    lo0 = c * HALF
    spill = HALF + s

    def ebody(j, carry):
        off = pl.multiple_of((s + j * NS) * CH, CH)
        pltpu.sync_copy(dst_hbm.at[pl.ds(off, CH)], draw)
        pltpu.sync_copy(src_hbm.at[pl.ds(off, CH)], sidx)
        cp = pltpu.async_copy(res_hbm.at[sidx], rbuf, sem)
        for k in range(CH // 16):
            v = draw[pl.ds(k * 16, 16)] - lo0
            ok = (v >= 0) & (v < HALF)
            dadj[pl.ds(k * 16, 16)] = jnp.where(ok, v, spill)
        cp.wait()
        pltpu.sync_copy(rbuf, shared.at[dadj], add=True)
        return carry

    lax.fori_loop(0, (N_ECH - s + NS - 1) // NS, ebody, 0)
    plsc.subcore_barrier()

    for q in range(TPR // CPY):
        b0 = s * TPR + q * CPY
        pltpu.sync_copy(shared.at[pl.ds(b0, CPY), :], zbuf)
        pltpu.sync_copy(zbuf, agg_out.at[pl.ds(c * HALF + b0, CPY), :])


def _elu(x):
    return jnp.where(x > 0, x, jnp.exp(jnp.minimum(x, 0.0)) - 1.0)


def _mlp3(x, w):
    h = _elu(jnp.dot(x, w[0][...], preferred_element_type=jnp.float32) + w[1][...])
    h = _elu(jnp.dot(h, w[2][...], preferred_element_type=jnp.float32) + w[3][...])
    return jnp.dot(h, w[4][...], preferred_element_type=jnp.float32) + w[5][...]


def _tc_body(gp, gs, op, ag, *rest):
    ws = rest[:30]
    out = rest[30]
    p = _mlp3(gp[...], ws[0:6])
    q = _mlp3(gs[...], ws[6:12])
    m = _mlp3(op[...], ws[12:18])
    a = _mlp3(ag[...], ws[18:24])
    o = _mlp3(jnp.concatenate([p, q, a, m], axis=-1), ws[24:30])
    i = pl.program_id(0)
    rows = i * R + lax.broadcasted_iota(jnp.int32, (R, 1), 0)
    keep = (rows >= 1) & (rows <= N_OPS - 2)
    out[...] = jnp.where(keep, o, 0.0)


def _full_spec(arr):
    return pl.BlockSpec(arr.shape, lambda i: (0,) * arr.ndim)


def _tc_call(gpred, gsucc, ops_pad, agg, ws):
    in_specs = [
        pl.BlockSpec((R, IN_C), lambda i: (i, 0)),
        pl.BlockSpec((R, IN_C), lambda i: (i, 0)),
        pl.BlockSpec((R, IN_C), lambda i: (i, 0)),
        pl.BlockSpec((R, OUT_C), lambda i: (i, 0)),
    ] + [_full_spec(w) for w in ws]
    return pl.pallas_call(
        _tc_body,
        grid=(NPAD // R,),
        in_specs=in_specs,
        out_specs=pl.BlockSpec((R, OUT_C), lambda i: (i, 0)),
        out_shape=jax.ShapeDtypeStruct((NPAD, OUT_C), jnp.float32),
        compiler_params=pltpu.CompilerParams(
            dimension_semantics=("arbitrary",)),
    )(gpred, gsucc, ops_pad, agg, *ws)


def kernel(operations, resources, requirement_edges, preds, succs, params):
    n = operations.shape[0]
    dst = requirement_edges[0].astype(jnp.int32)
    src = requirement_edges[1].astype(jnp.int32)
    preds_pad = jnp.pad(preds.astype(jnp.int32), (0, NPAD - n))
    succs_pad = jnp.pad(succs.astype(jnp.int32), (0, NPAD - n))

    agg, gpred, gsucc = _sc_call(
        resources, dst, src, operations, preds_pad, succs_pad)

    ops_pad = jnp.pad(operations, ((0, NPAD - n), (0, 0)))
    ws = []
    for nm in ("pred", "succ", "same", "res", "comb"):
        for j in range(3):
            ws.append(params[f"{nm}_W{j}"])
            ws.append(params[f"{nm}_b{j}"].reshape(1, -1))

    out = _tc_call(gpred, gsucc, ops_pad, agg, ws)
    return out[:n]


# speedup vs baseline: 3.1474x; 1.0005x over previous
import functools

import jax
import jax.numpy as jnp
from jax import lax
from jax.experimental import pallas as pl
from jax.experimental.pallas import tpu as pltpu
from jax.experimental.pallas import tpu_sc as plsc

N_OPS = 50000
E = 800000
IN_C = 128
OUT_C = 64
HID = 256

R = 512
NPAD = ((N_OPS + R - 1) // R) * R
HALF = NPAD // 2
CH = 128
GCH = 64
N_GCH = NPAD // GCH
N_ECH = E // CH
NC, NS = 2, 16
TPR = HALF // NS
CPY = 112
NDUMMY = 16

def _sc_call(*args):
    mesh = plsc.VectorSubcoreMesh(core_axis_name="c", subcore_axis_name="s")
    f = pl.kernel(
        _sc_body,
        mesh=mesh,
        out_type=(
            jax.ShapeDtypeStruct((NPAD, OUT_C), jnp.float32),
            jax.ShapeDtypeStruct((NPAD, IN_C), jnp.float32),
            jax.ShapeDtypeStruct((NPAD, IN_C), jnp.float32),
        ),
        scratch_types=[
            pltpu.VMEM((CH,), jnp.int32),
            pltpu.VMEM((CH,), jnp.int32),
            pltpu.VMEM((CH,), jnp.int32),
            pltpu.VMEM((CH, OUT_C), jnp.float32),
            pltpu.VMEM((GCH,), jnp.int32),
            pltpu.VMEM((GCH, IN_C), jnp.float32),
            pltpu.VMEM((CPY, OUT_C), jnp.float32),
            pltpu.VMEM_SHARED((HALF + NDUMMY, OUT_C), jnp.float32),
            pltpu.SemaphoreType.DMA,
        ],
        compiler_params=pltpu.CompilerParams(use_tc_tiling_on_sc=False),
    )
    return f(*args)


def _sc_body(res_hbm, dst_hbm, src_hbm, ops_hbm, preds_hbm, succs_hbm,
               agg_out, gpred_out, gsucc_out,
               draw, sidx, dadj, rbuf, ibuf, obuf, zbuf, shared, sem):
    c = lax.axis_index("c")
    s = lax.axis_index("s")
    gid = s * NC + c

    def _gather(idx_hbm, out_hbm):
        trips = (N_GCH - gid + NC * NS - 1) // (NC * NS)

        def body(j, carry):
            off = pl.multiple_of((gid + j * NC * NS) * GCH, GCH)
            pltpu.sync_copy(idx_hbm.at[pl.ds(off, GCH)], ibuf)
            pltpu.async_copy(ops_hbm.at[ibuf], obuf, sem).wait()
            pltpu.sync_copy(obuf, out_hbm.at[pl.ds(off, GCH), :])
            return carry

        lax.fori_loop(0, trips, body, 0)

    _gather(preds_hbm, gpred_out)
    _gather(succs_hbm, gsucc_out)

    def zrow(r, carry):
        for k in range(OUT_C // 16):
            zbuf[r, pl.ds(k * 16, 16)] = jnp.zeros((16,), jnp.float32)
        return carry

    lax.fori_loop(0, CPY, zrow, 0)
    for q in range(TPR // CPY):
        pltpu.sync_copy(zbuf, shared.at[pl.ds(s * TPR + q * CPY, CPY), :])
    plsc.subcore_barrier()

    lo0 = c * HALF
    spill = HALF + s

    def ebody(j, carry):
        off = pl.multiple_of((s + j * NS) * CH, CH)
        pltpu.sync_copy(dst_hbm.at[pl.ds(off, CH)], draw)
        pltpu.sync_copy(src_hbm.at[pl.ds(off, CH)], sidx)
        cp = pltpu.async_copy(res_hbm.at[sidx], rbuf, sem)
        for k in range(CH // 16):
            v = draw[pl.ds(k * 16, 16)] - lo0
            ok = (v >= 0) & (v < HALF)
            dadj[pl.ds(k * 16, 16)] = jnp.where(ok, v, spill)
        cp.wait()
        pltpu.sync_copy(rbuf, shared.at[dadj], add=True)
        return carry

    lax.fori_loop(0, (N_ECH - s + NS - 1) // NS, ebody, 0)
    plsc.subcore_barrier()

    for q in range(TPR // CPY):
        b0 = s * TPR + q * CPY
        pltpu.sync_copy(shared.at[pl.ds(b0, CPY), :], zbuf)
        pltpu.sync_copy(zbuf, agg_out.at[pl.ds(c * HALF + b0, CPY), :])


def _elu(x):
    return jnp.where(x > 0, x, jnp.exp(jnp.minimum(x, 0.0)) - 1.0)


def _mlp3(x, w):
    h = _elu(jnp.dot(x, w[0][...], preferred_element_type=jnp.float32) + w[1][...])
    h = _elu(jnp.dot(h, w[2][...], preferred_element_type=jnp.float32) + w[3][...])
    return jnp.dot(h, w[4][...], preferred_element_type=jnp.float32) + w[5][...]


def _tc_body(gp, gs, op, ag, *rest):
    ws = rest[:30]
    out = rest[30]
    p = _mlp3(gp[...], ws[0:6])
    q = _mlp3(gs[...], ws[6:12])
    m = _mlp3(op[...], ws[12:18])
    a = _mlp3(ag[...], ws[18:24])
    o = _mlp3(jnp.concatenate([p, q, a, m], axis=-1), ws[24:30])
    i = pl.program_id(0)
    rows = i * R + lax.broadcasted_iota(jnp.int32, (R, 1), 0)
    keep = (rows >= 1) & (rows <= N_OPS - 2)
    out[...] = jnp.where(keep, o, 0.0)


def _full_spec(arr):
    return pl.BlockSpec(arr.shape, lambda i: (0,) * arr.ndim)


def _tc_call(gpred, gsucc, ops_pad, agg, ws):
    in_specs = [
        pl.BlockSpec((R, IN_C), lambda i: (i, 0)),
        pl.BlockSpec((R, IN_C), lambda i: (i, 0)),
        pl.BlockSpec((R, IN_C), lambda i: (i, 0)),
        pl.BlockSpec((R, OUT_C), lambda i: (i, 0)),
    ] + [_full_spec(w) for w in ws]
    return pl.pallas_call(
        _tc_body,
        grid=(NPAD // R,),
        in_specs=in_specs,
        out_specs=pl.BlockSpec((R, OUT_C), lambda i: (i, 0)),
        out_shape=jax.ShapeDtypeStruct((NPAD, OUT_C), jnp.float32),
        compiler_params=pltpu.CompilerParams(
            dimension_semantics=("arbitrary",)),
    )(gpred, gsucc, ops_pad, agg, *ws)


def kernel(operations, resources, requirement_edges, preds, succs, params):
    n = operations.shape[0]
    dst = requirement_edges[0].astype(jnp.int32)
    src = requirement_edges[1].astype(jnp.int32)
    preds_pad = jnp.pad(preds.astype(jnp.int32), (0, NPAD - n))
    succs_pad = jnp.pad(succs.astype(jnp.int32), (0, NPAD - n))

    agg, gpred, gsucc = _sc_call(
        resources, dst, src, operations, preds_pad, succs_pad)

    ops_pad = jnp.pad(operations, ((0, NPAD - n), (0, 0)))
    ws = []
    for nm in ("pred", "succ", "same", "res", "comb"):
        for j in range(3):
            ws.append(params[f"{nm}_W{j}"])
            ws.append(params[f"{nm}_b{j}"].reshape(1, -1))

    out = _tc_call(gpred, gsucc, ops_pad, agg, ws)
    return out[:n]


# speedup vs baseline: 5.8935x; 1.8725x over previous
import functools

import jax
import jax.numpy as jnp
from jax import lax
from jax.experimental import pallas as pl
from jax.experimental.pallas import tpu as pltpu
from jax.experimental.pallas import tpu_sc as plsc

N_OPS = 50000
E = 800000
IN_C = 128
OUT_C = 64
HID = 256

R = 512
NPAD = ((N_OPS + R - 1) // R) * R
HALF = NPAD // 2
CH = 128
N_ECH = E // CH
NC, NS = 2, 16
TPR = HALF // NS
CPY = 56
SPILL = CH

IB = 10
CPS = N_ECH // NS
NBLK = CPS // IB
LEFT = N_ECH - CPS * NS

GR = 112
GPT = NPAD // (NC * NS)
NG = GPT // GR


def _mesh():
    return plsc.VectorSubcoreMesh(core_axis_name="c", subcore_axis_name="s")


def _gd_body(ops_hbm, preds_hbm, succs_hbm, gpred_out, gsucc_out,
             ibuf, ob0, ob1, sem0, sem1):
    c = lax.axis_index("c")
    s = lax.axis_index("s")
    gid = s * NC + c
    obs = (ob0, ob1)
    sems = (sem0, sem1)

    def _gather(idx_hbm, out_hbm):
        pltpu.sync_copy(idx_hbm.at[pl.ds(gid * NG, NG), :], ibuf)
        hs = [None, None]
        hs[0] = pltpu.async_copy(ops_hbm.at[ibuf.at[0]], ob0, sem0)
        for r in range(NG):
            if r + 1 < NG:
                hs[(r + 1) % 2] = pltpu.async_copy(
                    ops_hbm.at[ibuf.at[r + 1]], obs[(r + 1) % 2],
                    sems[(r + 1) % 2])
            hs[r % 2].wait()
            pltpu.sync_copy(obs[r % 2],
                            out_hbm.at[pl.ds(gid * GPT + r * GR, GR), :])

    _gather(preds_hbm, gpred_out)
    _gather(succs_hbm, gsucc_out)


def _gd_call(operations, preds2d, succs2d):
    f = pl.kernel(
        _gd_body,
        mesh=_mesh(),
        out_type=(
            jax.ShapeDtypeStruct((NPAD, IN_C), jnp.float32),
            jax.ShapeDtypeStruct((NPAD, IN_C), jnp.float32),
        ),
        scratch_types=[
            pltpu.VMEM((NG, GR), jnp.int32),
            pltpu.VMEM((GR, IN_C), jnp.float32),
            pltpu.VMEM((GR, IN_C), jnp.float32),
            pltpu.SemaphoreType.DMA,
            pltpu.SemaphoreType.DMA,
        ],
        compiler_params=pltpu.CompilerParams(use_tc_tiling_on_sc=False),
    )
    return f(operations, preds2d, succs2d)


def _sc_body(res_hbm, dst_hbm, src_hbm, agg_out,
             dbufA, dbufB, sbufA, sbufB, dadj, rbuf0, rbuf1, zbuf,
             shared, gsem0, gsem1, isemA, isemB):
    c = lax.axis_index("c")
    s = lax.axis_index("s")

    def zrow(r, carry):
        for k in range(OUT_C // 16):
            zbuf[r, pl.ds(k * 16, 16)] = jnp.zeros((16,), jnp.float32)
        return carry

    lax.fori_loop(0, CPY, zrow, 0)
    for q in range(TPR // CPY):
        pltpu.sync_copy(zbuf, shared.at[pl.ds(s * TPR + q * CPY, CPY), :])
    plsc.subcore_barrier()

    lo0 = c * HALF
    iota = lax.iota(jnp.int32, 16)
    ebase = s * CPS

    rbufs = (rbuf0, rbuf1)
    gsems = (gsem0, gsem1)

    def _issue_load(row0, dbufX, sbufX, isemX):
        pltpu.async_copy(dst_hbm.at[pl.ds(row0, IB), :], dbufX, isemX)
        pltpu.async_copy(src_hbm.at[pl.ds(row0, IB), :], sbufX, isemX)

    def _drain_load(dbufX, sbufX, isemX):
        pltpu.make_async_copy(dst_hbm.at[pl.ds(0, IB), :], dbufX, isemX).wait()
        pltpu.make_async_copy(src_hbm.at[pl.ds(0, IB), :], sbufX, isemX).wait()

    def _remap_scatter(dbufX, j, rb):
        for k in range(CH // 16):
            v = dbufX[j, pl.ds(k * 16, 16)] - lo0
            ok = (v >= 0) & (v < HALF)
            dadj[pl.ds(k * 16, 16)] = jnp.where(ok, v, HALF + k * 16 + iota)
        pltpu.sync_copy(rb, shared.at[dadj], add=True)

    def _process_block(dbufX, sbufX):
        hs = [None, None]
        hs[0] = pltpu.async_copy(res_hbm.at[sbufX.at[0]], rbufs[0], gsems[0])
        for j in range(IB):
            if j + 1 < IB:
                hs[(j + 1) % 2] = pltpu.async_copy(
                    res_hbm.at[sbufX.at[j + 1]], rbufs[(j + 1) % 2],
                    gsems[(j + 1) % 2])
            hs[j % 2].wait()
            _remap_scatter(dbufX, j, rbufs[j % 2])

    pltpu.sync_copy(dst_hbm.at[pl.ds(ebase, IB), :], dbufA)
    pltpu.sync_copy(src_hbm.at[pl.ds(ebase, IB), :], sbufA)
    _issue_load(ebase + IB, dbufB, sbufB, isemB)

    def blk_pair(i, carry):
        _process_block(dbufA, sbufA)
        _issue_load(ebase + (2 * i + 2) * IB, dbufA, sbufA, isemA)
        _drain_load(dbufB, sbufB, isemB)
        _process_block(dbufB, sbufB)
        _issue_load(ebase + (2 * i + 3) * IB, dbufB, sbufB, isemB)
        _drain_load(dbufA, sbufA, isemA)
        return carry

    lax.fori_loop(0, (NBLK - 3) // 2, blk_pair, 0)
    _process_block(dbufA, sbufA)
    _drain_load(dbufB, sbufB, isemB)
    _process_block(dbufB, sbufB)
    pltpu.sync_copy(dst_hbm.at[pl.ds(ebase + (NBLK - 1) * IB, IB), :], dbufA)
    pltpu.sync_copy(src_hbm.at[pl.ds(ebase + (NBLK - 1) * IB, IB), :], sbufA)
    _process_block(dbufA, sbufA)

    @pl.when(s < LEFT)
    def _leftover():
        row = CPS * NS + s
        pltpu.sync_copy(dst_hbm.at[pl.ds(row, 1), :], dbufB.at[pl.ds(0, 1), :])
        pltpu.sync_copy(src_hbm.at[pl.ds(row, 1), :], sbufB.at[pl.ds(0, 1), :])
        pltpu.async_copy(res_hbm.at[sbufB.at[0]], rbuf0, gsem0).wait()
        _remap_scatter(dbufB, 0, rbuf0)

    plsc.subcore_barrier()

    pltpu.sync_copy(shared.at[pl.ds(s * TPR, TPR), :],
                    agg_out.at[pl.ds(c * HALF + s * TPR, TPR), :])


def _sc_call(resources, dst2d, src2d):
    f = pl.kernel(
        _sc_body,
        mesh=_mesh(),
        out_type=jax.ShapeDtypeStruct((NPAD, OUT_C), jnp.float32),
        scratch_types=[
            pltpu.VMEM((IB, CH), jnp.int32),
            pltpu.VMEM((IB, CH), jnp.int32),
            pltpu.VMEM((IB, CH), jnp.int32),
            pltpu.VMEM((IB, CH), jnp.int32),
            pltpu.VMEM((CH,), jnp.int32),
            pltpu.VMEM((CH, OUT_C), jnp.float32),
            pltpu.VMEM((CH, OUT_C), jnp.float32),
            pltpu.VMEM((CPY, OUT_C), jnp.float32),
            pltpu.VMEM_SHARED((HALF + SPILL, OUT_C), jnp.float32),
            pltpu.SemaphoreType.DMA,
            pltpu.SemaphoreType.DMA,
            pltpu.SemaphoreType.DMA,
            pltpu.SemaphoreType.DMA,
        ],
        compiler_params=pltpu.CompilerParams(use_tc_tiling_on_sc=False),
    )
    return f(resources, dst2d, src2d)


def _elu(x):
    return jnp.where(x > 0, x, jnp.exp(jnp.minimum(x, 0.0)) - 1.0)


def _mlp3(x, w):
    h = _elu(jnp.dot(x, w[0][...], preferred_element_type=jnp.float32) + w[1][...])
    h = _elu(jnp.dot(h, w[2][...], preferred_element_type=jnp.float32) + w[3][...])
    return jnp.dot(h, w[4][...], preferred_element_type=jnp.float32) + w[5][...]


def _tc_body(gp, gs, op, ag, *rest):
    ws = rest[:30]
    out = rest[30]
    p = _mlp3(gp[...], ws[0:6])
    q = _mlp3(gs[...], ws[6:12])
    m = _mlp3(op[...], ws[12:18])
    a = _mlp3(ag[...], ws[18:24])
    o = _mlp3(jnp.concatenate([p, q, a, m], axis=-1), ws[24:30])
    i = pl.program_id(0)
    rows = i * R + lax.broadcasted_iota(jnp.int32, (R, 1), 0)
    keep = (rows >= 1) & (rows <= N_OPS - 2)
    out[...] = jnp.where(keep, o, 0.0)


def _full_spec(arr):
    return pl.BlockSpec(arr.shape, lambda i: (0,) * arr.ndim)


def _tc_call(gpred, gsucc, ops_pad, agg, ws):
    in_specs = [
        pl.BlockSpec((R, IN_C), lambda i: (i, 0)),
        pl.BlockSpec((R, IN_C), lambda i: (i, 0)),
        pl.BlockSpec((R, IN_C), lambda i: (i, 0)),
        pl.BlockSpec((R, OUT_C), lambda i: (i, 0)),
    ] + [_full_spec(w) for w in ws]
    return pl.pallas_call(
        _tc_body,
        grid=(NPAD // R,),
        in_specs=in_specs,
        out_specs=pl.BlockSpec((R, OUT_C), lambda i: (i, 0)),
        out_shape=jax.ShapeDtypeStruct((NPAD, OUT_C), jnp.float32),
        compiler_params=pltpu.CompilerParams(
            dimension_semantics=("arbitrary",)),
    )(gpred, gsucc, ops_pad, agg, *ws)


def kernel(operations, resources, requirement_edges, preds, succs, params):
    n = operations.shape[0]
    dst2d = requirement_edges[0].astype(jnp.int32).reshape(N_ECH, CH)
    src2d = requirement_edges[1].astype(jnp.int32).reshape(N_ECH, CH)
    preds2d = jnp.pad(preds.astype(jnp.int32), (0, NPAD - n)).reshape(-1, GR)
    succs2d = jnp.pad(succs.astype(jnp.int32), (0, NPAD - n)).reshape(-1, GR)

    gpred, gsucc = _gd_call(operations, preds2d, succs2d)
    agg = _sc_call(resources, dst2d, src2d)

    ops_pad = jnp.pad(operations, ((0, NPAD - n), (0, 0)))
    ws = []
    for nm in ("pred", "succ", "same", "res", "comb"):
        for j in range(3):
            ws.append(params[f"{nm}_W{j}"])
            ws.append(params[f"{nm}_b{j}"].reshape(1, -1))

    out = _tc_call(gpred, gsucc, ops_pad, agg, ws)
    return out[:n]


# speedup vs baseline: 6.7652x; 1.1479x over previous
import functools

import jax
import jax.numpy as jnp
from jax import lax
from jax.experimental import pallas as pl
from jax.experimental.pallas import tpu as pltpu
from jax.experimental.pallas import tpu_sc as plsc

N_OPS = 50000
E = 800000
IN_C = 128
OUT_C = 64
HID = 256

R = 512
NPAD = ((N_OPS + R - 1) // R) * R
HALF = NPAD // 2
CH = 128
N_ECH = E // CH
NC, NS = 2, 16
TPR = HALF // NS
CPY = 56
SPILL = CH

IB = 10
CPS = N_ECH // NS
NBLK = CPS // IB
LEFT = N_ECH - CPS * NS

GR = 112
GPT = NPAD // (NC * NS)
NG = GPT // GR


def _mesh():
    return plsc.VectorSubcoreMesh(core_axis_name="c", subcore_axis_name="s")


def _gd_body(ops_hbm, preds_hbm, succs_hbm, gpred_out, gsucc_out,
             ibuf, ob0, ob1, sem0, sem1):
    c = lax.axis_index("c")
    s = lax.axis_index("s")
    gid = s * NC + c
    obs = (ob0, ob1)
    sems = (sem0, sem1)

    def _gather(idx_hbm, out_hbm):
        pltpu.sync_copy(idx_hbm.at[pl.ds(gid * NG, NG), :], ibuf)
        hs = [None, None]
        hs[0] = pltpu.async_copy(ops_hbm.at[ibuf.at[0]], ob0, sem0)
        for r in range(NG):
            if r + 1 < NG:
                hs[(r + 1) % 2] = pltpu.async_copy(
                    ops_hbm.at[ibuf.at[r + 1]], obs[(r + 1) % 2],
                    sems[(r + 1) % 2])
            hs[r % 2].wait()
            pltpu.sync_copy(obs[r % 2],
                            out_hbm.at[pl.ds(gid * GPT + r * GR, GR), :])

    _gather(preds_hbm, gpred_out)
    _gather(succs_hbm, gsucc_out)


def _gd_call(operations, preds2d, succs2d):
    f = pl.kernel(
        _gd_body,
        mesh=_mesh(),
        out_type=(
            jax.ShapeDtypeStruct((NPAD, IN_C), jnp.float32),
            jax.ShapeDtypeStruct((NPAD, IN_C), jnp.float32),
        ),
        scratch_types=[
            pltpu.VMEM((NG, GR), jnp.int32),
            pltpu.VMEM((GR, IN_C), jnp.float32),
            pltpu.VMEM((GR, IN_C), jnp.float32),
            pltpu.SemaphoreType.DMA,
            pltpu.SemaphoreType.DMA,
        ],
        compiler_params=pltpu.CompilerParams(use_tc_tiling_on_sc=False),
    )
    return f(operations, preds2d, succs2d)


def _sc_body(res_hbm, dst_hbm, src_hbm, agg_out,
             dbufA, dbufB, sbufA, sbufB, dadj, rbuf0, rbuf1, zbuf,
             shared, gsem0, gsem1, isemA, isemB):
    c = lax.axis_index("c")
    s = lax.axis_index("s")

    def zrow(r, carry):
        for k in range(OUT_C // 16):
            zbuf[r, pl.ds(k * 16, 16)] = jnp.zeros((16,), jnp.float32)
        return carry

    lax.fori_loop(0, CPY, zrow, 0)
    for q in range(TPR // CPY):
        pltpu.sync_copy(zbuf, shared.at[pl.ds(s * TPR + q * CPY, CPY), :])
    plsc.subcore_barrier()

    lo0 = c * HALF
    iota = lax.iota(jnp.int32, 16)
    ebase = s * CPS

    rbufs = (rbuf0, rbuf1)
    gsems = (gsem0, gsem1)

    def _issue_load(row0, dbufX, sbufX, isemX):
        pltpu.async_copy(dst_hbm.at[pl.ds(row0, IB), :], dbufX, isemX)
        pltpu.async_copy(src_hbm.at[pl.ds(row0, IB), :], sbufX, isemX)

    def _drain_load(dbufX, sbufX, isemX):
        pltpu.make_async_copy(dst_hbm.at[pl.ds(0, IB), :], dbufX, isemX).wait()
        pltpu.make_async_copy(src_hbm.at[pl.ds(0, IB), :], sbufX, isemX).wait()

    def _remap_scatter(dbufX, j, rb):
        for k in range(CH // 16):
            v = dbufX[j, pl.ds(k * 16, 16)] - lo0
            ok = (v >= 0) & (v < HALF)
            dadj[pl.ds(k * 16, 16)] = jnp.where(ok, v, HALF + k * 16 + iota)
        pltpu.sync_copy(rb, shared.at[dadj], add=True)

    def _process_block(dbufX, sbufX):
        hs = [None, None]
        hs[0] = pltpu.async_copy(res_hbm.at[sbufX.at[0]], rbufs[0], gsems[0])
        for j in range(IB):
            if j + 1 < IB:
                hs[(j + 1) % 2] = pltpu.async_copy(
                    res_hbm.at[sbufX.at[j + 1]], rbufs[(j + 1) % 2],
                    gsems[(j + 1) % 2])
            hs[j % 2].wait()
            _remap_scatter(dbufX, j, rbufs[j % 2])

    pltpu.sync_copy(dst_hbm.at[pl.ds(ebase, IB), :], dbufA)
    pltpu.sync_copy(src_hbm.at[pl.ds(ebase, IB), :], sbufA)
    _issue_load(ebase + IB, dbufB, sbufB, isemB)

    def blk_pair(i, carry):
        _process_block(dbufA, sbufA)
        _issue_load(ebase + (2 * i + 2) * IB, dbufA, sbufA, isemA)
        _drain_load(dbufB, sbufB, isemB)
        _process_block(dbufB, sbufB)
        _issue_load(ebase + (2 * i + 3) * IB, dbufB, sbufB, isemB)
        _drain_load(dbufA, sbufA, isemA)
        return carry

    lax.fori_loop(0, (NBLK - 3) // 2, blk_pair, 0)
    _process_block(dbufA, sbufA)
    _drain_load(dbufB, sbufB, isemB)
    _process_block(dbufB, sbufB)
    pltpu.sync_copy(dst_hbm.at[pl.ds(ebase + (NBLK - 1) * IB, IB), :], dbufA)
    pltpu.sync_copy(src_hbm.at[pl.ds(ebase + (NBLK - 1) * IB, IB), :], sbufA)
    _process_block(dbufA, sbufA)

    @pl.when(s < LEFT)
    def _leftover():
        row = CPS * NS + s
        pltpu.sync_copy(dst_hbm.at[pl.ds(row, 1), :], dbufB.at[pl.ds(0, 1), :])
        pltpu.sync_copy(src_hbm.at[pl.ds(row, 1), :], sbufB.at[pl.ds(0, 1), :])
        pltpu.async_copy(res_hbm.at[sbufB.at[0]], rbuf0, gsem0).wait()
        _remap_scatter(dbufB, 0, rbuf0)

    plsc.subcore_barrier()

    pltpu.sync_copy(shared.at[pl.ds(s * TPR, TPR), :],
                    agg_out.at[pl.ds(c * HALF + s * TPR, TPR), :])


def _sc_call(resources, dst2d, src2d):
    f = pl.kernel(
        _sc_body,
        mesh=_mesh(),
        out_type=jax.ShapeDtypeStruct((NPAD, OUT_C), jnp.float32),
        scratch_types=[
            pltpu.VMEM((IB, CH), jnp.int32),
            pltpu.VMEM((IB, CH), jnp.int32),
            pltpu.VMEM((IB, CH), jnp.int32),
            pltpu.VMEM((IB, CH), jnp.int32),
            pltpu.VMEM((CH,), jnp.int32),
            pltpu.VMEM((CH, OUT_C), jnp.float32),
            pltpu.VMEM((CH, OUT_C), jnp.float32),
            pltpu.VMEM((CPY, OUT_C), jnp.float32),
            pltpu.VMEM_SHARED((HALF + SPILL, OUT_C), jnp.float32),
            pltpu.SemaphoreType.DMA,
            pltpu.SemaphoreType.DMA,
            pltpu.SemaphoreType.DMA,
            pltpu.SemaphoreType.DMA,
        ],
        compiler_params=pltpu.CompilerParams(use_tc_tiling_on_sc=False),
    )
    return f(resources, dst2d, src2d)


def _elu(x):
    return jnp.where(x > 0, x, jnp.exp(jnp.minimum(x, 0.0)) - 1.0)


def _mlp3(x, w):
    h = _elu(jnp.dot(x, w[0][...], preferred_element_type=jnp.float32) + w[1][...])
    h = _elu(jnp.dot(h, w[2][...], preferred_element_type=jnp.float32) + w[3][...])
    return jnp.dot(h, w[4][...], preferred_element_type=jnp.float32) + w[5][...]


def _full_spec(arr):
    return pl.BlockSpec(arr.shape, lambda i: (0,) * arr.ndim)


def _row_spec(cols):
    return pl.BlockSpec((R, cols), lambda i: (i, 0))


def _tc_same_body(op, *rest):
    ws = rest[:6]
    out = rest[6]
    out[...] = _mlp3(op[...], ws)


def _tc_same_call(operations, ws):
    n = operations.shape[0]
    return pl.pallas_call(
        _tc_same_body,
        grid=(NPAD // R,),
        in_specs=[_row_spec(IN_C)] + [_full_spec(w) for w in ws],
        out_specs=_row_spec(OUT_C),
        out_shape=jax.ShapeDtypeStruct((n, OUT_C), jnp.float32),
        compiler_params=pltpu.CompilerParams(
            dimension_semantics=("arbitrary",)),
    )(operations, *ws)


def _tc_ps_body(gp, gs, *rest):
    ws = rest[:12]
    pout, qout = rest[12], rest[13]
    pout[...] = _mlp3(gp[...], ws[0:6])
    qout[...] = _mlp3(gs[...], ws[6:12])


def _tc_ps_call(gpred, gsucc, ws):
    return pl.pallas_call(
        _tc_ps_body,
        grid=(NPAD // R,),
        in_specs=[_row_spec(IN_C), _row_spec(IN_C)]
        + [_full_spec(w) for w in ws],
        out_specs=(_row_spec(OUT_C), _row_spec(OUT_C)),
        out_shape=(
            jax.ShapeDtypeStruct((NPAD, OUT_C), jnp.float32),
            jax.ShapeDtypeStruct((NPAD, OUT_C), jnp.float32),
        ),
        compiler_params=pltpu.CompilerParams(
            dimension_semantics=("arbitrary",)),
    )(gpred, gsucc, *ws)


def _tc_comb_body(p, q, m, ag, *rest):
    ws = rest[:12]
    out = rest[12]
    a = _mlp3(ag[...], ws[0:6])
    o = _mlp3(jnp.concatenate([p[...], q[...], a, m[...]], axis=-1), ws[6:12])
    i = pl.program_id(0)
    rows = i * R + lax.broadcasted_iota(jnp.int32, (R, 1), 0)
    keep = (rows >= 1) & (rows <= N_OPS - 2)
    out[...] = jnp.where(keep, o, 0.0)


def _tc_comb_call(p, q, m, agg, ws, n):
    return pl.pallas_call(
        _tc_comb_body,
        grid=(NPAD // R,),
        in_specs=[_row_spec(OUT_C)] * 4 + [_full_spec(w) for w in ws],
        out_specs=_row_spec(OUT_C),
        out_shape=jax.ShapeDtypeStruct((n, OUT_C), jnp.float32),
        compiler_params=pltpu.CompilerParams(
            dimension_semantics=("arbitrary",)),
    )(p, q, m, agg, *ws)


def kernel(operations, resources, requirement_edges, preds, succs, params):
    n = operations.shape[0]
    dst2d = requirement_edges[0].astype(jnp.int32).reshape(N_ECH, CH)
    src2d = requirement_edges[1].astype(jnp.int32).reshape(N_ECH, CH)
    preds2d = jnp.pad(preds.astype(jnp.int32), (0, NPAD - n)).reshape(-1, GR)
    succs2d = jnp.pad(succs.astype(jnp.int32), (0, NPAD - n)).reshape(-1, GR)

    def w6(nm):
        out = []
        for j in range(3):
            out.append(params[f"{nm}_W{j}"])
            out.append(params[f"{nm}_b{j}"].reshape(1, -1))
        return out

    gpred, gsucc = _gd_call(operations, preds2d, succs2d)
    agg = _sc_call(resources, dst2d, src2d)

    m = _tc_same_call(operations, w6("same"))
    p, q = _tc_ps_call(gpred, gsucc, w6("pred") + w6("succ"))
    return _tc_comb_call(p, q, m, agg, w6("res") + w6("comb"), n)


# speedup vs baseline: 7.3939x; 1.0929x over previous
import functools

import jax
import jax.numpy as jnp
from jax import lax
from jax.experimental import pallas as pl
from jax.experimental.pallas import tpu as pltpu
from jax.experimental.pallas import tpu_sc as plsc

N_OPS = 50000
E = 800000
IN_C = 128
OUT_C = 64
HID = 256

R = 512
NPAD = ((N_OPS + R - 1) // R) * R
HALF = NPAD // 2
CH = 128
N_ECH = E // CH
NC, NS = 2, 16
TPR = HALF // NS
CPY = 56
SPILL = CH

IB = 10
CPS = N_ECH // NS
NBLK = CPS // IB
LEFT = N_ECH - CPS * NS

GR = 112
GPT = NPAD // (NC * NS)
NG = GPT // GR


def _mesh():
    return plsc.VectorSubcoreMesh(core_axis_name="c", subcore_axis_name="s")


def _gd_body(ops_hbm, preds_hbm, succs_hbm, gpred_out, gsucc_out,
             ibuf, ob0, ob1, sem0, sem1):
    c = lax.axis_index("c")
    s = lax.axis_index("s")
    gid = s * NC + c
    obs = (ob0, ob1)
    sems = (sem0, sem1)

    def _gather(idx_hbm, out_hbm):
        pltpu.sync_copy(idx_hbm.at[pl.ds(gid * NG, NG), :], ibuf)
        hs = [None, None]
        hs[0] = pltpu.async_copy(ops_hbm.at[ibuf.at[0]], ob0, sem0)
        for r in range(NG):
            if r + 1 < NG:
                hs[(r + 1) % 2] = pltpu.async_copy(
                    ops_hbm.at[ibuf.at[r + 1]], obs[(r + 1) % 2],
                    sems[(r + 1) % 2])
            hs[r % 2].wait()
            pltpu.sync_copy(obs[r % 2],
                            out_hbm.at[pl.ds(gid * GPT + r * GR, GR), :])

    _gather(preds_hbm, gpred_out)
    _gather(succs_hbm, gsucc_out)


def _gd_call(operations, preds2d, succs2d):
    f = pl.kernel(
        _gd_body,
        mesh=_mesh(),
        out_type=(
            jax.ShapeDtypeStruct((NPAD, IN_C), jnp.float32),
            jax.ShapeDtypeStruct((NPAD, IN_C), jnp.float32),
        ),
        scratch_types=[
            pltpu.VMEM((NG, GR), jnp.int32),
            pltpu.VMEM((GR, IN_C), jnp.float32),
            pltpu.VMEM((GR, IN_C), jnp.float32),
            pltpu.SemaphoreType.DMA,
            pltpu.SemaphoreType.DMA,
        ],
        compiler_params=pltpu.CompilerParams(use_tc_tiling_on_sc=False),
    )
    return f(operations, preds2d, succs2d)


COLS = OUT_C // NC
ZR = 112
RPT = NPAD // NS


def _sc_body(res_hbm, dst_hbm, src_hbm, agg_out,
             dbufA, dbufB, sbufA, sbufB, rbuf0, rbuf1, zbuf,
             shared, gsem0, gsem1, isemA, isemB):
    c = lax.axis_index("c")
    s = lax.axis_index("s")
    col0 = pl.multiple_of(c * COLS, COLS)
    res_c = res_hbm.at[c]

    def zrow(r, carry):
        for k in range(COLS // 16):
            zbuf[r, pl.ds(k * 16, 16)] = jnp.zeros((16,), jnp.float32)
        return carry

    lax.fori_loop(0, ZR, zrow, 0)
    for q in range(RPT // ZR):
        pltpu.sync_copy(zbuf, shared.at[pl.ds(s * RPT + q * ZR, ZR), :])
    plsc.subcore_barrier()

    ebase = s * CPS

    rbufs = (rbuf0, rbuf1)
    gsems = (gsem0, gsem1)

    def _issue_load(row0, dbufX, sbufX, isemX):
        pltpu.async_copy(dst_hbm.at[pl.ds(row0, IB), :], dbufX, isemX)
        pltpu.async_copy(src_hbm.at[pl.ds(row0, IB), :], sbufX, isemX)

    def _drain_load(dbufX, sbufX, isemX):
        pltpu.make_async_copy(dst_hbm.at[pl.ds(0, IB), :], dbufX, isemX).wait()
        pltpu.make_async_copy(src_hbm.at[pl.ds(0, IB), :], sbufX, isemX).wait()

    def _process_block(dbufX, sbufX):
        hs = [None, None]
        hs[0] = pltpu.async_copy(res_c.at[sbufX.at[0]], rbufs[0], gsems[0])
        for j in range(IB):
            if j + 1 < IB:
                hs[(j + 1) % 2] = pltpu.async_copy(
                    res_c.at[sbufX.at[j + 1]],
                    rbufs[(j + 1) % 2], gsems[(j + 1) % 2])
            hs[j % 2].wait()
            pltpu.sync_copy(rbufs[j % 2], shared.at[dbufX.at[j]], add=True)

    pltpu.sync_copy(dst_hbm.at[pl.ds(ebase, IB), :], dbufA)
    pltpu.sync_copy(src_hbm.at[pl.ds(ebase, IB), :], sbufA)
    _issue_load(ebase + IB, dbufB, sbufB, isemB)

    def blk_pair(i, carry):
        _process_block(dbufA, sbufA)
        _issue_load(ebase + (2 * i + 2) * IB, dbufA, sbufA, isemA)
        _drain_load(dbufB, sbufB, isemB)
        _process_block(dbufB, sbufB)
        _issue_load(ebase + (2 * i + 3) * IB, dbufB, sbufB, isemB)
        _drain_load(dbufA, sbufA, isemA)
        return carry

    lax.fori_loop(0, (NBLK - 3) // 2, blk_pair, 0)
    _process_block(dbufA, sbufA)
    _drain_load(dbufB, sbufB, isemB)
    _process_block(dbufB, sbufB)
    pltpu.sync_copy(dst_hbm.at[pl.ds(ebase + (NBLK - 1) * IB, IB), :], dbufA)
    pltpu.sync_copy(src_hbm.at[pl.ds(ebase + (NBLK - 1) * IB, IB), :], sbufA)
    _process_block(dbufA, sbufA)

    @pl.when(s < LEFT)
    def _leftover():
        row = CPS * NS + s
        pltpu.sync_copy(dst_hbm.at[pl.ds(row, 1), :], dbufB.at[pl.ds(0, 1), :])
        pltpu.sync_copy(src_hbm.at[pl.ds(row, 1), :], sbufB.at[pl.ds(0, 1), :])
        pltpu.async_copy(res_c.at[sbufB.at[0]], rbuf0, gsem0).wait()
        pltpu.sync_copy(rbuf0, shared.at[dbufB.at[0]], add=True)

    plsc.subcore_barrier()

    pltpu.sync_copy(shared.at[pl.ds(s * RPT, RPT), :],
                    agg_out.at[pl.ds(s * RPT, RPT), pl.ds(col0, COLS)])


def _sc_call(resources, dst2d, src2d):
    f = pl.kernel(
        _sc_body,
        mesh=_mesh(),
        out_type=jax.ShapeDtypeStruct((NPAD, OUT_C), jnp.float32),
        scratch_types=[
            pltpu.VMEM((IB, CH), jnp.int32),
            pltpu.VMEM((IB, CH), jnp.int32),
            pltpu.VMEM((IB, CH), jnp.int32),
            pltpu.VMEM((IB, CH), jnp.int32),
            pltpu.VMEM((CH, COLS), jnp.float32),
            pltpu.VMEM((CH, COLS), jnp.float32),
            pltpu.VMEM((ZR, COLS), jnp.float32),
            pltpu.VMEM_SHARED((NPAD, COLS), jnp.float32),
            pltpu.SemaphoreType.DMA,
            pltpu.SemaphoreType.DMA,
            pltpu.SemaphoreType.DMA,
            pltpu.SemaphoreType.DMA,
        ],
        compiler_params=pltpu.CompilerParams(use_tc_tiling_on_sc=False),
    )
    return f(resources, dst2d, src2d)


def _split_res(resources):
    return jnp.stack(
        [resources[:, :COLS], resources[:, COLS:]])


def _elu(x):
    return jnp.where(x > 0, x, jnp.exp(jnp.minimum(x, 0.0)) - 1.0)


def _mlp3(x, w):
    h = _elu(jnp.dot(x, w[0][...], preferred_element_type=jnp.float32) + w[1][...])
    h = _elu(jnp.dot(h, w[2][...], preferred_element_type=jnp.float32) + w[3][...])
    return jnp.dot(h, w[4][...], preferred_element_type=jnp.float32) + w[5][...]


def _full_spec(arr):
    return pl.BlockSpec(arr.shape, lambda i: (0,) * arr.ndim)


def _row_spec(cols):
    return pl.BlockSpec((R, cols), lambda i: (i, 0))


def _tc_same_body(op, *rest):
    ws = rest[:6]
    out = rest[6]
    out[...] = _mlp3(op[...], ws)


def _tc_same_call(operations, ws):
    n = operations.shape[0]
    return pl.pallas_call(
        _tc_same_body,
        grid=(NPAD // R,),
        in_specs=[_row_spec(IN_C)] + [_full_spec(w) for w in ws],
        out_specs=_row_spec(OUT_C),
        out_shape=jax.ShapeDtypeStruct((n, OUT_C), jnp.float32),
        compiler_params=pltpu.CompilerParams(
            dimension_semantics=("arbitrary",)),
    )(operations, *ws)


def _tc_ps_body(gp, gs, *rest):
    ws = rest[:12]
    pout, qout = rest[12], rest[13]
    pout[...] = _mlp3(gp[...], ws[0:6])
    qout[...] = _mlp3(gs[...], ws[6:12])


def _tc_ps_call(gpred, gsucc, ws):
    return pl.pallas_call(
        _tc_ps_body,
        grid=(NPAD // R,),
        in_specs=[_row_spec(IN_C), _row_spec(IN_C)]
        + [_full_spec(w) for w in ws],
        out_specs=(_row_spec(OUT_C), _row_spec(OUT_C)),
        out_shape=(
            jax.ShapeDtypeStruct((NPAD, OUT_C), jnp.float32),
            jax.ShapeDtypeStruct((NPAD, OUT_C), jnp.float32),
        ),
        compiler_params=pltpu.CompilerParams(
            dimension_semantics=("arbitrary",)),
    )(gpred, gsucc, *ws)


def _tc_comb_body(p, q, m, ag, *rest):
    ws = rest[:12]
    out = rest[12]
    a = _mlp3(ag[...], ws[0:6])
    o = _mlp3(jnp.concatenate([p[...], q[...], a, m[...]], axis=-1), ws[6:12])
    i = pl.program_id(0)
    rows = i * R + lax.broadcasted_iota(jnp.int32, (R, 1), 0)
    keep = (rows >= 1) & (rows <= N_OPS - 2)
    out[...] = jnp.where(keep, o, 0.0)


def _tc_comb_call(p, q, m, agg, ws, n):
    return pl.pallas_call(
        _tc_comb_body,
        grid=(NPAD // R,),
        in_specs=[_row_spec(OUT_C)] * 4 + [_full_spec(w) for w in ws],
        out_specs=_row_spec(OUT_C),
        out_shape=jax.ShapeDtypeStruct((n, OUT_C), jnp.float32),
        compiler_params=pltpu.CompilerParams(
            dimension_semantics=("arbitrary",)),
    )(p, q, m, agg, *ws)


def kernel(operations, resources, requirement_edges, preds, succs, params):
    n = operations.shape[0]
    dst2d = requirement_edges[0].astype(jnp.int32).reshape(N_ECH, CH)
    src2d = requirement_edges[1].astype(jnp.int32).reshape(N_ECH, CH)
    preds2d = jnp.pad(preds.astype(jnp.int32), (0, NPAD - n)).reshape(-1, GR)
    succs2d = jnp.pad(succs.astype(jnp.int32), (0, NPAD - n)).reshape(-1, GR)

    def w6(nm):
        out = []
        for j in range(3):
            out.append(params[f"{nm}_W{j}"])
            out.append(params[f"{nm}_b{j}"].reshape(1, -1))
        return out

    gpred, gsucc = _gd_call(operations, preds2d, succs2d)
    agg = _sc_call(_split_res(resources), dst2d, src2d)

    m = _tc_same_call(operations, w6("same"))
    p, q = _tc_ps_call(gpred, gsucc, w6("pred") + w6("succ"))
    return _tc_comb_call(p, q, m, agg, w6("res") + w6("comb"), n)
